# baseline (speedup 1.0000x reference)
import jax
import jax.numpy as jnp
from jax.experimental import pallas as pl
from jax.experimental.pallas import tpu as pltpu

_B, _P, _C = 32, 8732, 21
_R = _P // 4
_EPS = 1e-07


def _isd_body(lam_ref, conf_ref, temp_ref, interp_ref, loc_ref, loct_ref,
              loci_ref, out_i_ref, out_f_ref, acc_ref):
    b = pl.program_id(0)

    @pl.when(b == 0)
    def _init():
        acc_ref[...] = jnp.zeros_like(acc_ref)

    lam = lam_ref[0, 0]
    conf_t = jnp.swapaxes(conf_ref[0], 0, 1)
    temp_t = jnp.swapaxes(temp_ref[0], 0, 1)
    interp_t = jnp.swapaxes(interp_ref[0], 0, 1)
    loc_t = jnp.swapaxes(loc_ref[0], 0, 1)
    loct_t = jnp.swapaxes(loct_ref[0], 0, 1)
    loci_t = jnp.swapaxes(loci_ref[0], 0, 1)

    for k in range(4):
        conf = conf_t[21 * k:21 * k + 21, :]
        temp = temp_t[21 * k:21 * k + 21, :]
        interp = interp_t[21 * k:21 * k + 21, :] + _EPS

        left = (jnp.max(conf[1:, :], axis=0, keepdims=True)
                > conf[0:1, :]).astype(jnp.float32)
        right = (jnp.max(temp[1:, :], axis=0, keepdims=True)
                 > temp[0:1, :]).astype(jnp.float32)
        inter = left * right
        only_l = left * (1.0 - right)
        only_r = right * (1.0 - left)

        mixed = lam * conf + (1.0 - lam) * temp + _EPS
        conf_eps = conf + _EPS
        temp_eps = temp + _EPS
        log_mixed = jnp.log(mixed)
        log_interp = jnp.log(interp)
        log_conf = jnp.log(conf_eps)
        log_temp = jnp.log(temp_eps)

        kl_ab = jnp.sum((interp - mixed) * (log_interp - log_mixed),
                        axis=0, keepdims=True)
        kl_l = jnp.sum(conf_eps * (log_conf - log_interp),
                       axis=0, keepdims=True)
        kl_r = jnp.sum(temp_eps * (log_temp - log_interp),
                       axis=0, keepdims=True)

        dl = loci_t[4 * k:4 * k + 4, :] - loc_t[4 * k:4 * k + 4, :]
        dr = loci_t[4 * k:4 * k + 4, :] - loct_t[4 * k:4 * k + 4, :]
        se_l = jnp.sum(dl * dl, axis=0, keepdims=True)
        se_r = jnp.sum(dr * dr, axis=0, keepdims=True)

        acc_ref[0:1, :] += kl_ab * inter
        acc_ref[1:2, :] += inter
        acc_ref[2:3, :] += (kl_l + 0.25 * se_l) * only_l
        acc_ref[3:4, :] += only_l
        acc_ref[4:5, :] += (kl_r + 0.25 * se_r) * only_r
        acc_ref[5:6, :] += only_r

    @pl.when(b == _B - 1)
    def _finish():
        s_ab = jnp.sum(acc_ref[0:1, :])
        cnt_i = jnp.sum(acc_ref[1:2, :])
        s_l = jnp.sum(acc_ref[2:3, :])
        cnt_l = jnp.sum(acc_ref[3:4, :])
        s_r = jnp.sum(acc_ref[4:5, :])
        cnt_r = jnp.sum(acc_ref[5:6, :])
        interp_loss = jnp.where(cnt_i > 0.0,
                                s_ab / (2.0 * jnp.maximum(cnt_i, 1.0)), 0.0)
        fix_loss = (jnp.where(cnt_l > 0.0, s_l / jnp.maximum(cnt_l, 1.0), 0.0)
                    + jnp.where(cnt_r > 0.0, s_r / jnp.maximum(cnt_r, 1.0), 0.0))
        out_i_ref[0, 0] = interp_loss
        out_f_ref[0, 0] = fix_loss


def _conf_spec(swap):
    if swap:
        return pl.BlockSpec((1, _R, 84), lambda b: ((b + _B // 2) % _B, 0, 0))
    return pl.BlockSpec((1, _R, 84), lambda b: (b, 0, 0))


def _loc_spec(swap):
    if swap:
        return pl.BlockSpec((1, _R, 16), lambda b: ((b + _B // 2) % _B, 0, 0))
    return pl.BlockSpec((1, _R, 16), lambda b: (b, 0, 0))


def kernel(lam, conf, conf_flip, loc, loc_flip, conf_shuffle,
           conf_interpolation, loc_shuffle, loc_interpolation):
    del conf_flip, loc_flip
    lam2d = jnp.reshape(lam.astype(jnp.float32), (1, 1))

    out_i, out_f = pl.pallas_call(
        _isd_body,
        grid=(_B,),
        in_specs=[
            pl.BlockSpec(memory_space=pltpu.SMEM),
            _conf_spec(False),
            _conf_spec(True),
            _conf_spec(False),
            _loc_spec(False),
            _loc_spec(True),
            _loc_spec(False),
        ],
        out_specs=[
            pl.BlockSpec(memory_space=pltpu.SMEM),
            pl.BlockSpec(memory_space=pltpu.SMEM),
        ],
        out_shape=[
            jax.ShapeDtypeStruct((1, 1), jnp.float32),
            jax.ShapeDtypeStruct((1, 1), jnp.float32),
        ],
        scratch_shapes=[pltpu.VMEM((8, _R), jnp.float32)],
    )(lam2d,
      conf.reshape(_B, _R, 84), conf_shuffle.reshape(_B, _R, 84),
      conf_interpolation.reshape(_B, _R, 84),
      loc.reshape(_B, _R, 16), loc_shuffle.reshape(_B, _R, 16),
      loc_interpolation.reshape(_B, _R, 16))
    return out_i.reshape(()), out_f.reshape(())


# speedup vs baseline: 5.3640x; 5.3640x over previous
import jax
import jax.numpy as jnp
from jax.experimental import pallas as pl
from jax.experimental.pallas import tpu as pltpu

_B, _P, _C = 32, 8732, 21
_EPS = 1e-07


def _isd_body(lam_ref, conf_ref, temp_ref, interp_ref, loc_ref, loct_ref,
              loci_ref, out_i_ref, out_f_ref,
              a0_ref, a1_ref, a2_ref, a3_ref, a4_ref, a5_ref):
    b = pl.program_id(0)

    @pl.when(b == 0)
    def _init():
        for r in (a0_ref, a1_ref, a2_ref, a3_ref, a4_ref, a5_ref):
            r[...] = jnp.zeros_like(r)

    lam = lam_ref[0, 0]
    conf = conf_ref[0]
    temp = temp_ref[0]
    interp = interp_ref[0] + _EPS

    left = (jnp.max(conf, axis=0, keepdims=True)
            > conf[0:1, :]).astype(jnp.float32)
    right = (jnp.max(temp, axis=0, keepdims=True)
             > temp[0:1, :]).astype(jnp.float32)
    inter = left * right
    only_l = left * (1.0 - right)
    only_r = right * (1.0 - left)

    mixed = lam * conf + (1.0 - lam) * temp + _EPS
    conf_eps = conf + _EPS
    temp_eps = temp + _EPS
    log_mixed = jnp.log(mixed)
    log_interp = jnp.log(interp)
    log_conf = jnp.log(conf_eps)
    log_temp = jnp.log(temp_eps)

    kl_ab = jnp.sum((interp - mixed) * (log_interp - log_mixed),
                    axis=0, keepdims=True)
    kl_l = jnp.sum(conf_eps * (log_conf - log_interp), axis=0, keepdims=True)
    kl_r = jnp.sum(temp_eps * (log_temp - log_interp), axis=0, keepdims=True)

    dl = loci_ref[0] - loc_ref[0]
    dr = loci_ref[0] - loct_ref[0]
    se_l = jnp.sum(dl * dl, axis=0, keepdims=True)
    se_r = jnp.sum(dr * dr, axis=0, keepdims=True)

    a0_ref[...] += kl_ab * inter
    a1_ref[...] += inter
    a2_ref[...] += (kl_l + 0.25 * se_l) * only_l
    a3_ref[...] += only_l
    a4_ref[...] += (kl_r + 0.25 * se_r) * only_r
    a5_ref[...] += only_r

    @pl.when(b == _B - 1)
    def _finish():
        s_ab = jnp.sum(a0_ref[...])
        cnt_i = jnp.sum(a1_ref[...])
        s_l = jnp.sum(a2_ref[...])
        cnt_l = jnp.sum(a3_ref[...])
        s_r = jnp.sum(a4_ref[...])
        cnt_r = jnp.sum(a5_ref[...])
        interp_loss = jnp.where(cnt_i > 0.0,
                                s_ab / (2.0 * jnp.maximum(cnt_i, 1.0)), 0.0)
        fix_loss = (jnp.where(cnt_l > 0.0, s_l / jnp.maximum(cnt_l, 1.0), 0.0)
                    + jnp.where(cnt_r > 0.0, s_r / jnp.maximum(cnt_r, 1.0), 0.0))
        out_i_ref[0, 0] = interp_loss
        out_f_ref[0, 0] = fix_loss


def _conf_spec(swap):
    if swap:
        return pl.BlockSpec((1, _C, _P), lambda b: ((b + _B // 2) % _B, 0, 0))
    return pl.BlockSpec((1, _C, _P), lambda b: (b, 0, 0))


def _loc_spec(swap):
    if swap:
        return pl.BlockSpec((1, 4, _P), lambda b: ((b + _B // 2) % _B, 0, 0))
    return pl.BlockSpec((1, 4, _P), lambda b: (b, 0, 0))


def kernel(lam, conf, conf_flip, loc, loc_flip, conf_shuffle,
           conf_interpolation, loc_shuffle, loc_interpolation):
    del conf_flip, loc_flip
    conf_t = jnp.swapaxes(conf, 1, 2)
    shuf_t = jnp.swapaxes(conf_shuffle, 1, 2)
    interp_t = jnp.swapaxes(conf_interpolation, 1, 2)
    loc_t = jnp.swapaxes(loc, 1, 2)
    locs_t = jnp.swapaxes(loc_shuffle, 1, 2)
    loci_t = jnp.swapaxes(loc_interpolation, 1, 2)
    lam2d = jnp.reshape(lam.astype(jnp.float32), (1, 1))

    out_i, out_f = pl.pallas_call(
        _isd_body,
        grid=(_B,),
        in_specs=[
            pl.BlockSpec(memory_space=pltpu.SMEM),
            _conf_spec(False),
            _conf_spec(True),
            _conf_spec(False),
            _loc_spec(False),
            _loc_spec(True),
            _loc_spec(False),
        ],
        out_specs=[
            pl.BlockSpec(memory_space=pltpu.SMEM),
            pl.BlockSpec(memory_space=pltpu.SMEM),
        ],
        out_shape=[
            jax.ShapeDtypeStruct((1, 1), jnp.float32),
            jax.ShapeDtypeStruct((1, 1), jnp.float32),
        ],
        scratch_shapes=[pltpu.VMEM((1, _P), jnp.float32)] * 6,
    )(lam2d, conf_t, shuf_t, interp_t, loc_t, locs_t, loci_t)
    return out_i.reshape(()), out_f.reshape(())
